# Initial kernel scaffold; baseline (speedup 1.0000x reference)
import jax
import jax.numpy as jnp
from jax.experimental import pallas as pl


def kernel(hidden_states, router_weight, gate_proj, up_proj, down_proj):
    raise NotImplementedError("write your pallas kernel here")


# speedup vs baseline: 1.3263x; 1.3263x over previous
import jax
import jax.numpy as jnp
from jax import lax
from jax.experimental import pallas as pl
from jax.experimental.pallas import tpu as pltpu

NUM_EXPERTS = 64
TOP_K = 8
HIDDEN = 1024
FF = 768
T = 128


def _moe_body(x_ref, rw_ref, g_ref, u_ref, d_ref, out_ref, logits_ref, comb_ref):
    e = pl.program_id(0)

    @pl.when(e == 0)
    def _router():
        x = x_ref[...]
        lg = lax.dot_general(x, rw_ref[...], (((1,), (1,)), ((), ())),
                             preferred_element_type=jnp.float32)
        logits_ref[...] = lg
        colid = lax.broadcasted_iota(jnp.int32, (T, NUM_EXPERTS), 1)
        remaining = lg
        sel = jnp.zeros((T, NUM_EXPERTS), jnp.float32)
        for _ in range(TOP_K):
            m = jnp.max(remaining, axis=1, keepdims=True)
            is_m = remaining == m
            first_col = jnp.min(jnp.where(is_m, colid, NUM_EXPERTS),
                                axis=1, keepdims=True)
            pick = colid == first_col
            sel = jnp.where(pick, 1.0, sel)
            remaining = jnp.where(pick, -3.0e38, remaining)
        mx = jnp.max(lg, axis=1, keepdims=True)
        ex = sel * jnp.exp(lg - mx)
        comb_ref[...] = ex / jnp.sum(ex, axis=1, keepdims=True)

    x = x_ref[...]
    g = lax.dot_general(x, g_ref[0], (((1,), (1,)), ((), ())),
                        preferred_element_type=jnp.float32)
    u = lax.dot_general(x, u_ref[0], (((1,), (1,)), ((), ())),
                        preferred_element_type=jnp.float32)
    h = g * (1.0 / (1.0 + jnp.exp(-g))) * u
    onehot = (lax.broadcasted_iota(jnp.int32, (NUM_EXPERTS, 1), 0) == e
              ).astype(jnp.float32)
    col = lax.dot_general(comb_ref[...], onehot, (((1,), (0,)), ((), ())),
                          preferred_element_type=jnp.float32)
    h = h * col
    contrib = lax.dot_general(h, d_ref[0], (((1,), (1,)), ((), ())),
                              preferred_element_type=jnp.float32)

    @pl.when(e == 0)
    def _init():
        out_ref[...] = contrib

    @pl.when(e > 0)
    def _acc():
        out_ref[...] += contrib


def kernel(hidden_states, router_weight, gate_proj, up_proj, down_proj):
    B, S, D = hidden_states.shape
    x = hidden_states.reshape(-1, D)

    out, logits = pl.pallas_call(
        _moe_body,
        grid=(NUM_EXPERTS,),
        in_specs=[
            pl.BlockSpec((T, HIDDEN), lambda e: (0, 0)),
            pl.BlockSpec((NUM_EXPERTS, HIDDEN), lambda e: (0, 0)),
            pl.BlockSpec((1, FF, HIDDEN), lambda e: (e, 0, 0)),
            pl.BlockSpec((1, FF, HIDDEN), lambda e: (e, 0, 0)),
            pl.BlockSpec((1, HIDDEN, FF), lambda e: (e, 0, 0)),
        ],
        out_specs=[
            pl.BlockSpec((T, HIDDEN), lambda e: (0, 0)),
            pl.BlockSpec((T, NUM_EXPERTS), lambda e: (0, 0)),
        ],
        out_shape=[
            jax.ShapeDtypeStruct((T, HIDDEN), jnp.float32),
            jax.ShapeDtypeStruct((T, NUM_EXPERTS), jnp.float32),
        ],
        scratch_shapes=[pltpu.VMEM((T, NUM_EXPERTS), jnp.float32)],
    )(x, router_weight, gate_proj, up_proj, down_proj)

    return out.reshape(B, S, D), logits
